# baseline (speedup 1.0000x reference)
import functools

import jax
import jax.numpy as jnp
from jax import lax
from jax.experimental import pallas as pl
from jax.experimental.pallas import tpu as pltpu
from jax.experimental.pallas import tpu_sc as plsc

NC = 2
NS = 16
NW = NC * NS

DIM = 64


def _sc_gather(titems, cidx, nidx, tvectors, cvectors):
    B = titems.shape[0]
    F = cidx.shape[0]
    t_per_w = B // NW
    f_per_w = F // NW
    CHUNK = 1280
    n_chunks = f_per_w // CHUNK

    mesh = plsc.VectorSubcoreMesh(core_axis_name="c", subcore_axis_name="s")

    @functools.partial(
        pl.kernel,
        mesh=mesh,
        compiler_params=pltpu.CompilerParams(use_tc_tiling_on_sc=False),
        out_type=(
            jax.ShapeDtypeStruct((B, DIM), jnp.float32),
            jax.ShapeDtypeStruct((F, DIM), jnp.float32),
            jax.ShapeDtypeStruct((F, DIM), jnp.float32),
        ),
        scratch_types=[
            pltpu.VMEM((t_per_w,), jnp.int32),
            pltpu.VMEM((f_per_w,), jnp.int32),
            pltpu.VMEM((f_per_w,), jnp.int32),
            pltpu.VMEM((CHUNK, DIM), jnp.float32),
            pltpu.SemaphoreType.DMA,
        ],
    )
    def gather_kernel(tit_h, cit_h, nit_h, tv_h, cv_h,
                      tout_h, cout_h, nout_h,
                      idx_t, idx_c, idx_n, rows, sem):
        wid = lax.axis_index("s") * NC + lax.axis_index("c")
        tb = wid * t_per_w
        fb = wid * f_per_w
        pltpu.sync_copy(tit_h.at[pl.ds(tb, t_per_w)], idx_t)
        pltpu.sync_copy(cit_h.at[pl.ds(fb, f_per_w)], idx_c)
        pltpu.sync_copy(nit_h.at[pl.ds(fb, f_per_w)], idx_n)
        pltpu.async_copy(tv_h.at[idx_t], rows.at[pl.ds(0, t_per_w)], sem).wait()
        pltpu.sync_copy(rows.at[pl.ds(0, t_per_w)], tout_h.at[pl.ds(tb, t_per_w)])
        for ch in range(n_chunks):
            pltpu.async_copy(cv_h.at[idx_c.at[pl.ds(ch * CHUNK, CHUNK)]],
                             rows, sem).wait()
            pltpu.sync_copy(rows, cout_h.at[pl.ds(fb + ch * CHUNK, CHUNK)])
        for ch in range(n_chunks):
            pltpu.async_copy(tv_h.at[idx_n.at[pl.ds(ch * CHUNK, CHUNK)]],
                             rows, sem).wait()
            pltpu.sync_copy(rows, nout_h.at[pl.ds(fb + ch * CHUNK, CHUNK)])

    return gather_kernel(titems, cidx, nidx, tvectors, cvectors)


def _tc_loss(tvecs, cvecs, nvecs, ctx, negs):
    B = tvecs.shape[0]
    NB = 8
    steps = B // NB
    R = NB * ctx
    C = NB + NB * negs

    def body(tv_ref, cv_ref, nv_ref, out_ref):
        cv = cv_ref[...]
        allt = jnp.concatenate([tv_ref[...], nv_ref[...]], axis=0)
        g = lax.dot_general(cv, allt, (((1,), (1,)), ((), ())),
                            preferred_element_type=jnp.float32)
        row_b = lax.broadcasted_iota(jnp.int32, (R, C), 0) // ctx
        col = lax.broadcasted_iota(jnp.int32, (R, C), 1)
        is_t = col < NB
        col_b = jnp.where(is_t, col, (col - NB) // negs)
        mask = row_b == col_b
        x = jnp.where(is_t, -g, g)
        sp = jnp.maximum(x, 0.0) + jnp.log1p(jnp.exp(-jnp.abs(x)))
        part = jnp.sum(jnp.where(mask, sp, 0.0), keepdims=True)

        @pl.when(pl.program_id(0) == 0)
        def _():
            out_ref[...] = jnp.zeros((1, 1), jnp.float32)

        out_ref[...] += part

    out = pl.pallas_call(
        body,
        grid=(steps,),
        in_specs=[
            pl.BlockSpec((NB, DIM), lambda i: (i, 0)),
            pl.BlockSpec((R, DIM), lambda i: (i, 0)),
            pl.BlockSpec((R, DIM), lambda i: (i, 0)),
        ],
        out_specs=pl.BlockSpec((1, 1), lambda i: (0, 0)),
        out_shape=jax.ShapeDtypeStruct((1, 1), jnp.float32),
    )(tvecs, cvecs, nvecs)
    return out[0, 0]


def kernel(titems, citems, nitems, tvectors, cvectors):
    B, ctx = citems.shape
    negs = nitems.shape[1]
    tvecs, cvecs, nvecs = _sc_gather(
        titems, citems.reshape(-1), nitems.reshape(-1), tvectors, cvectors)
    total = _tc_loss(tvecs, cvecs, nvecs, ctx, negs)
    return total / B
